# baseline (speedup 1.0000x reference)
import functools

import jax
import jax.numpy as jnp
from jax import lax
from jax.experimental import pallas as pl
from jax.experimental.pallas import tpu as pltpu
from jax.experimental.pallas import tpu_sc as plsc

BATCH = 4096
SEQ_LEN = 200
EMBED_DIM = 64

_info = plsc.get_sparse_core_info()
NC, NS = _info.num_cores, _info.num_subcores
NW = NC * NS
B_PER_W = BATCH // NW
NBUF = 4
PF = 3


def _gather_kernel(table_hbm, idx_hbm, out_hbm,
                   idx_v, rows_v, blk_v,
                   gs0, gs1, gs2, gs3, os0, os1):
    gsem = (gs0, gs1, gs2, gs3)
    osem = (os0, os1)
    wid = lax.axis_index("s") * NC + lax.axis_index("c")
    b0 = wid * B_PER_W

    pltpu.sync_copy(idx_hbm.at[:, pl.ds(b0, B_PER_W)], idx_v)

    def gather_desc(l, k):
        return pltpu.make_async_copy(
            table_hbm.at[idx_v.at[l]], rows_v.at[k], gsem[k]
        )

    def oc_desc(l, kb):
        return pltpu.make_async_copy(
            blk_v.at[kb], out_hbm.at[l, :, pl.ds(wid, 1)], osem[kb]
        )

    def transpose(l, kin, kout):
        del l
        rv = rows_v.at[kin]

        def jbody(jg, _):
            j0 = jg * 16
            jvec = lax.iota(jnp.int32, 16) + j0

            def rbody(R, _):
                for r in range(8):
                    d = 8 * R + r
                    dvec = jnp.full((16,), 0, jnp.int32) + d
                    vals = plsc.load_gather(rv, [jvec, dvec])
                    blk_v[kout, R, 0, r, pl.ds(j0, 16)] = vals
                return ()

            lax.fori_loop(0, 8, rbody, ())
            return ()

        lax.fori_loop(0, 8, jbody, ())

    def step(l, k, kb, first=False, pf=True):
        gather_desc(l, k).wait()
        if pf:
            gather_desc(l + PF, (k + PF) % NBUF).start()
        transpose(l, k, kb)
        if not first:
            oc_desc(l - 1, 1 - kb).wait()
        oc_desc(l, kb).start()

    for l0 in range(PF):
        gather_desc(l0, l0).start()
    step(0, 0, 0, first=True)
    step(1, 1, 1)
    step(2, 2, 0)
    step(3, 3, 1)

    def body(t, _):
        l = 4 * t
        step(l, 0, 0)
        step(l + 1, 1, 1)
        step(l + 2, 2, 0)
        step(l + 3, 3, 1)
        return ()

    lax.fori_loop(1, (SEQ_LEN - 4) // 4, body, ())

    step(196, 0, 0)
    step(197, 1, 1, pf=False)
    step(198, 2, 0, pf=False)
    step(199, 3, 1, pf=False)
    oc_desc(199, 1).wait()


@jax.jit
def _run(table, idx_t):
    mesh = plsc.VectorSubcoreMesh(core_axis_name="c", subcore_axis_name="s")
    fn = functools.partial(
        pl.kernel,
        mesh=mesh,
        out_type=jax.ShapeDtypeStruct((SEQ_LEN, 8, NW, 8, 128), jnp.float32),
        scratch_types=[
            pltpu.VMEM((SEQ_LEN, B_PER_W), jnp.int32),
            pltpu.VMEM((NBUF, B_PER_W, EMBED_DIM), jnp.float32),
            pltpu.VMEM((2, 8, 1, 8, 128), jnp.float32),
            pltpu.SemaphoreType.DMA,
            pltpu.SemaphoreType.DMA,
            pltpu.SemaphoreType.DMA,
            pltpu.SemaphoreType.DMA,
            pltpu.SemaphoreType.DMA,
            pltpu.SemaphoreType.DMA,
        ],
        compiler_params=pltpu.CompilerParams(
            use_tc_tiling_on_sc=False,
            needs_layout_passes=False,
            disable_bounds_checks=True,
        ),
    )(_gather_kernel)
    return fn(table, idx_t)


def kernel(indices, table):
    idx_t = jnp.swapaxes(indices, 0, 1).astype(jnp.int32)
    out = _run(table, idx_t)
    return jnp.transpose(out, (2, 4, 0, 1, 3)).reshape(BATCH, SEQ_LEN, EMBED_DIM)
